# baseline (speedup 1.0000x reference)
import functools

import jax
import jax.numpy as jnp
from jax import lax
from jax.experimental import pallas as pl
from jax.experimental.pallas import tpu as pltpu
from jax.experimental.pallas import tpu_sc as plsc

BATCH = 4096
SEQ = 200
D = 128
NC = 2
NS = 16
NW = NC * NS
B_PER_W = BATCH // NW
LANES = 16
SEQ_A = 128
SEQ_B = SEQ - SEQ_A


def _pool_body(tok, tab, out, idx_a, idx_b, rows, pooled, sem):
    wid = lax.axis_index("s") * NC + lax.axis_index("c")
    base = wid * B_PER_W

    def row_body(i, carry):
        row = base + i
        pltpu.sync_copy(tok.at[row, pl.ds(0, SEQ_A)], idx_a)
        pltpu.sync_copy(tok.at[row, pl.ds(SEQ_A, SEQ_B)], idx_b)
        ca = pltpu.async_copy(tab.at[idx_a], rows.at[pl.ds(0, SEQ_A)], sem)
        cb = pltpu.async_copy(tab.at[idx_b], rows.at[pl.ds(SEQ_A, SEQ_B)], sem)
        ca.wait()
        cb.wait()

        def seq_body(s, accs):
            return tuple(
                accs[d] + rows[s, pl.ds(LANES * d, LANES)]
                for d in range(D // LANES)
            )

        accs = lax.fori_loop(
            0, SEQ, seq_body,
            tuple(jnp.zeros((LANES,), jnp.float32) for _ in range(D // LANES)),
        )
        for d in range(D // LANES):
            pooled[i, pl.ds(LANES * d, LANES)] = accs[d] * (1.0 / SEQ)
        return carry

    lax.fori_loop(0, B_PER_W, row_body, 0)
    pltpu.sync_copy(pooled, out.at[pl.ds(base, B_PER_W)])


_pool = pl.kernel(
    _pool_body,
    out_type=jax.ShapeDtypeStruct((BATCH, D), jnp.float32),
    mesh=plsc.VectorSubcoreMesh(core_axis_name="c", subcore_axis_name="s"),
    scratch_types=[
        pltpu.VMEM((SEQ_A,), jnp.int32),
        pltpu.VMEM((SEQ_B,), jnp.int32),
        pltpu.VMEM((SEQ, D), jnp.float32),
        pltpu.VMEM((B_PER_W, D), jnp.float32),
        pltpu.SemaphoreType.DMA,
    ],
)


def _proj_body(p_ref, w_ref, b_ref, o_ref):
    o_ref[:] = (
        lax.dot_general(
            p_ref[:], w_ref[:],
            (((1,), (1,)), ((), ())),
            preferred_element_type=jnp.float32,
        )
        + b_ref[:]
    )


_proj = pl.pallas_call(
    _proj_body,
    out_shape=jax.ShapeDtypeStruct((BATCH, D), jnp.float32),
)


@jax.jit
def kernel(token_ids, embedding, W, b):
    token_ids = token_ids.astype(jnp.int32)
    pooled = _pool(token_ids, embedding)
    return _proj(pooled, W, b.reshape(1, D))


# speedup vs baseline: 2.3387x; 2.3387x over previous
import functools

import jax
import jax.numpy as jnp
from jax import lax
from jax.experimental import pallas as pl
from jax.experimental.pallas import tpu as pltpu
from jax.experimental.pallas import tpu_sc as plsc

BATCH = 4096
SEQ = 200
D = 128
NC = 2
NS = 16
NW = NC * NS
B_PER_W = BATCH // NW
LANES = 16
SEQ_A = 128
SEQ_B = SEQ - SEQ_A


def _pool_body(tok_a, tok_b, tab, out, idx_a, idx_b, rows0, rows1, pooled,
               sem):
    wid = lax.axis_index("s") * NC + lax.axis_index("c")
    base = wid * B_PER_W

    pltpu.sync_copy(tok_a.at[pl.ds(base, B_PER_W)], idx_a)
    pltpu.sync_copy(tok_b.at[pl.ds(base, B_PER_W)], idx_b)

    def _gather(i, buf):
        return (
            pltpu.make_async_copy(tab.at[idx_a.at[i]],
                                  buf.at[pl.ds(0, SEQ_A)], sem),
            pltpu.make_async_copy(tab.at[idx_b.at[i]],
                                  buf.at[pl.ds(SEQ_A, SEQ_B)], sem),
        )

    def issue(i, buf):
        ca, cb = _gather(i, buf)
        ca.start()
        cb.start()

    def drain(i, buf):
        ca, cb = _gather(i, buf)
        ca.wait()
        cb.wait()

    def accum(i, buf):
        def seq_body(s, accs):
            return tuple(
                accs[d] + buf[s, pl.ds(LANES * d, LANES)]
                for d in range(D // LANES)
            )

        accs = lax.fori_loop(
            0, SEQ, seq_body,
            tuple(jnp.zeros((LANES,), jnp.float32) for _ in range(D // LANES)),
        )
        for d in range(D // LANES):
            pooled[i, pl.ds(LANES * d, LANES)] = accs[d] * (1.0 / SEQ)

    issue(0, rows0)

    def pair_body(p, carry):
        i = 2 * p
        issue(i + 1, rows1)
        drain(i, rows0)
        accum(i, rows0)

        @pl.when(p < B_PER_W // 2 - 1)
        def _():
            issue(i + 2, rows0)

        drain(i + 1, rows1)
        accum(i + 1, rows1)
        return carry

    lax.fori_loop(0, B_PER_W // 2, pair_body, 0)
    pltpu.sync_copy(pooled, out.at[pl.ds(base, B_PER_W)])


_pool = pl.kernel(
    _pool_body,
    out_type=jax.ShapeDtypeStruct((BATCH, D), jnp.float32),
    mesh=plsc.VectorSubcoreMesh(core_axis_name="c", subcore_axis_name="s"),
    scratch_types=[
        pltpu.VMEM((B_PER_W, SEQ_A), jnp.int32),
        pltpu.VMEM((B_PER_W, SEQ_B), jnp.int32),
        pltpu.VMEM((SEQ, D), jnp.float32),
        pltpu.VMEM((SEQ, D), jnp.float32),
        pltpu.VMEM((B_PER_W, D), jnp.float32),
        pltpu.SemaphoreType.DMA,
    ],
)


def _proj_body(p_ref, w_ref, b_ref, o_ref):
    o_ref[:] = (
        lax.dot_general(
            p_ref[:], w_ref[:],
            (((1,), (1,)), ((), ())),
            preferred_element_type=jnp.float32,
        )
        + b_ref[:]
    )


_proj = pl.pallas_call(
    _proj_body,
    out_shape=jax.ShapeDtypeStruct((BATCH, D), jnp.float32),
)


@jax.jit
def kernel(token_ids, embedding, W, b):
    token_ids = token_ids.astype(jnp.int32)
    tok_a = token_ids[:, :SEQ_A]
    tok_b = token_ids[:, SEQ_A:]
    pooled = _pool(tok_a, tok_b, embedding)
    return _proj(pooled, W, b.reshape(1, D))
